# baseline (speedup 1.0000x reference)
import jax
import jax.numpy as jnp
from jax.experimental import pallas as pl
from jax.experimental.pallas import tpu as pltpu

_SIZES = (8, 8, 16, 16, 16, 16, 16, 16, 8, 8)


def _dma_copy_kernel(x_ref, o_ref, *refs):
    n_chunks = len(_SIZES)
    bufs = refs[:n_chunks]
    in_sems, out_sems = refs[n_chunks], refs[n_chunks + 1]
    offs = [0]
    for s in _SIZES:
        offs.append(offs[-1] + s)

    def in_copy(c):
        return pltpu.make_async_copy(
            x_ref.at[pl.ds(offs[c], _SIZES[c]), :], bufs[c], in_sems.at[c]
        )

    def out_copy(c):
        return pltpu.make_async_copy(
            bufs[c], o_ref.at[pl.ds(offs[c], _SIZES[c]), :], out_sems.at[c]
        )

    for c in range(n_chunks):
        in_copy(c).start()
    for c in range(n_chunks):
        in_copy(c).wait()
        out_copy(c).start()
    for c in range(n_chunks):
        out_copy(c).wait()


def kernel(x, scale_p, scale_n):
    del scale_p, scale_n
    m, n = x.shape
    out = pl.pallas_call(
        _dma_copy_kernel,
        in_specs=[pl.BlockSpec(memory_space=pl.ANY)],
        out_specs=pl.BlockSpec(memory_space=pl.ANY),
        out_shape=jax.ShapeDtypeStruct((m, n), x.dtype),
        scratch_shapes=[pltpu.VMEM((s, n), x.dtype) for s in _SIZES]
        + [
            pltpu.SemaphoreType.DMA((len(_SIZES),)),
            pltpu.SemaphoreType.DMA((len(_SIZES),)),
        ],
    )(x)
    return out


# speedup vs baseline: 1.0202x; 1.0202x over previous
import jax
import jax.numpy as jnp
from jax.experimental import pallas as pl
from jax.experimental.pallas import tpu as pltpu

_SIZES = (8, 32, 40, 40, 8)


def _dma_copy_kernel(x_ref, o_ref, *refs):
    n_chunks = len(_SIZES)
    bufs = refs[:n_chunks]
    in_sems, out_sems = refs[n_chunks], refs[n_chunks + 1]
    offs = [0]
    for s in _SIZES:
        offs.append(offs[-1] + s)

    def in_copy(c):
        return pltpu.make_async_copy(
            x_ref.at[pl.ds(offs[c], _SIZES[c]), :], bufs[c], in_sems.at[c]
        )

    def out_copy(c):
        return pltpu.make_async_copy(
            bufs[c], o_ref.at[pl.ds(offs[c], _SIZES[c]), :], out_sems.at[c]
        )

    for c in range(n_chunks):
        in_copy(c).start()
    for c in range(n_chunks):
        in_copy(c).wait()
        out_copy(c).start()
    for c in range(n_chunks):
        out_copy(c).wait()


def kernel(x, scale_p, scale_n):
    del scale_p, scale_n
    m, n = x.shape
    out = pl.pallas_call(
        _dma_copy_kernel,
        in_specs=[pl.BlockSpec(memory_space=pl.ANY)],
        out_specs=pl.BlockSpec(memory_space=pl.ANY),
        out_shape=jax.ShapeDtypeStruct((m, n), x.dtype),
        scratch_shapes=[pltpu.VMEM((s, n), x.dtype) for s in _SIZES]
        + [
            pltpu.SemaphoreType.DMA((len(_SIZES),)),
            pltpu.SemaphoreType.DMA((len(_SIZES),)),
        ],
    )(x)
    return out


# speedup vs baseline: 1.0368x; 1.0163x over previous
import jax
import jax.numpy as jnp
from jax.experimental import pallas as pl
from jax.experimental.pallas import tpu as pltpu

_SIZES = (8, 16, 24, 32, 24, 16, 8)


def _dma_copy_kernel(x_ref, o_ref, *refs):
    n_chunks = len(_SIZES)
    bufs = refs[:n_chunks]
    in_sems, out_sems = refs[n_chunks], refs[n_chunks + 1]
    offs = [0]
    for s in _SIZES:
        offs.append(offs[-1] + s)

    def in_copy(c):
        return pltpu.make_async_copy(
            x_ref.at[pl.ds(offs[c], _SIZES[c]), :], bufs[c], in_sems.at[c]
        )

    def out_copy(c):
        return pltpu.make_async_copy(
            bufs[c], o_ref.at[pl.ds(offs[c], _SIZES[c]), :], out_sems.at[c]
        )

    for c in range(n_chunks):
        in_copy(c).start()
    for c in range(n_chunks):
        in_copy(c).wait()
        out_copy(c).start()
    for c in range(n_chunks):
        out_copy(c).wait()


def kernel(x, scale_p, scale_n):
    del scale_p, scale_n
    m, n = x.shape
    out = pl.pallas_call(
        _dma_copy_kernel,
        in_specs=[pl.BlockSpec(memory_space=pl.ANY)],
        out_specs=pl.BlockSpec(memory_space=pl.ANY),
        out_shape=jax.ShapeDtypeStruct((m, n), x.dtype),
        scratch_shapes=[pltpu.VMEM((s, n), x.dtype) for s in _SIZES]
        + [
            pltpu.SemaphoreType.DMA((len(_SIZES),)),
            pltpu.SemaphoreType.DMA((len(_SIZES),)),
        ],
    )(x)
    return out


# speedup vs baseline: 1.0594x; 1.0218x over previous
import jax
import jax.numpy as jnp
from jax.experimental import pallas as pl
from jax.experimental.pallas import tpu as pltpu

_SIZES = (8, 24, 32, 32, 16, 8, 8)


def _dma_copy_kernel(x_ref, o_ref, *refs):
    n_chunks = len(_SIZES)
    bufs = refs[:n_chunks]
    in_sems, out_sems = refs[n_chunks], refs[n_chunks + 1]
    offs = [0]
    for s in _SIZES:
        offs.append(offs[-1] + s)

    def in_copy(c):
        return pltpu.make_async_copy(
            x_ref.at[pl.ds(offs[c], _SIZES[c]), :], bufs[c], in_sems.at[c]
        )

    def out_copy(c):
        return pltpu.make_async_copy(
            bufs[c], o_ref.at[pl.ds(offs[c], _SIZES[c]), :], out_sems.at[c]
        )

    for c in range(n_chunks):
        in_copy(c).start()
    for c in range(n_chunks):
        in_copy(c).wait()
        out_copy(c).start()
    for c in range(n_chunks):
        out_copy(c).wait()


def kernel(x, scale_p, scale_n):
    del scale_p, scale_n
    m, n = x.shape
    out = pl.pallas_call(
        _dma_copy_kernel,
        in_specs=[pl.BlockSpec(memory_space=pl.ANY)],
        out_specs=pl.BlockSpec(memory_space=pl.ANY),
        out_shape=jax.ShapeDtypeStruct((m, n), x.dtype),
        scratch_shapes=[pltpu.VMEM((s, n), x.dtype) for s in _SIZES]
        + [
            pltpu.SemaphoreType.DMA((len(_SIZES),)),
            pltpu.SemaphoreType.DMA((len(_SIZES),)),
        ],
    )(x)
    return out
